# Initial kernel scaffold; baseline (speedup 1.0000x reference)
import jax
import jax.numpy as jnp
from jax.experimental import pallas as pl


def kernel(x, edge_index, W1, b1, W2, b2):
    raise NotImplementedError("write your pallas kernel here")


# speedup vs baseline: 29.3192x; 29.3192x over previous
import functools

import jax
import jax.numpy as jnp
from jax import lax
from jax.experimental import pallas as pl
from jax.experimental.pallas import tpu as pltpu
from jax.experimental.pallas import tpu_sc as plsc

N_NODES = 10000
N_EDGES = 320000
D_FEAT = 128
HIDDEN = 16

NC = 2
NS = 16
NW = NC * NS
BLK = 128
NBLK = 79
EPT = BLK * NBLK
E_PAD = EPT * NW
ACC_N = 10112
RPT = ACC_N // NS
PAD_SPREAD = 100


def _mesh():
    return plsc.VectorSubcoreMesh(core_axis_name="c", subcore_axis_name="s")


def _deg_kernel(dst_hbm, zeros_hbm, deg_out, dst_v, ones_v, wb_v, deg_sh):
    c = lax.axis_index("c")
    s = lax.axis_index("s")
    wid = c * NS + s
    pltpu.sync_copy(zeros_hbm.at[pl.ds(s * RPT, RPT)], wb_v)
    pltpu.sync_copy(wb_v, deg_sh.at[pl.ds(s * RPT, RPT)])
    for k in range(BLK // 16):
        ones_v[pl.ds(k * 16, 16)] = jnp.ones((16,), jnp.float32)
    pltpu.sync_copy(dst_hbm.at[wid], dst_v)
    plsc.subcore_barrier()

    def body(j, carry):
        pltpu.sync_copy(ones_v, deg_sh.at[dst_v.at[j]], add=True)
        return carry

    lax.fori_loop(0, NBLK, body, 0)
    plsc.subcore_barrier()
    pltpu.sync_copy(deg_sh.at[pl.ds(s * RPT, RPT)], wb_v)
    pltpu.sync_copy(wb_v, deg_out.at[pl.ds(c * ACC_N + s * RPT, RPT)])


@functools.partial(jax.jit, donate_argnums=())
def _deg_pass(dst_w, zeros1):
    return pl.kernel(
        _deg_kernel,
        out_type=jax.ShapeDtypeStruct((NC * ACC_N,), jnp.float32),
        mesh=_mesh(),
        scratch_types=[
            pltpu.VMEM((NBLK, BLK), jnp.int32),
            pltpu.VMEM((BLK,), jnp.float32),
            pltpu.VMEM((RPT,), jnp.float32),
            pltpu.VMEM_SHARED((ACC_N,), jnp.float32),
        ],
    )(dst_w, zeros1)


def _feat_kernel(src_hbm, dst_hbm, htt_hbm, zeros_hbm, s1_out,
                 src_v, dst_v, tab_v, upd_v, wb_v, sf_sh):
    c = lax.axis_index("c")
    s = lax.axis_index("s")
    wid = c * NS + s
    pltpu.sync_copy(src_hbm.at[wid], src_v)
    pltpu.sync_copy(dst_hbm.at[wid], dst_v)
    for f in range(HIDDEN):
        pltpu.sync_copy(zeros_hbm.at[pl.ds(s * RPT, RPT)], wb_v)
        pltpu.sync_copy(wb_v, sf_sh.at[pl.ds(s * RPT, RPT)])
        pltpu.sync_copy(htt_hbm.at[pl.ds(f * ACC_N, ACC_N)], tab_v)
        plsc.subcore_barrier()

        def body(j, carry):
            for k in range(BLK // 16):
                idx = src_v[j, pl.ds(k * 16, 16)]
                upd_v[pl.ds(k * 16, 16)] = plsc.load_gather(tab_v, [idx])
            pltpu.sync_copy(upd_v, sf_sh.at[dst_v.at[j]], add=True)
            return carry

        lax.fori_loop(0, NBLK, body, 0)
        plsc.subcore_barrier()
        pltpu.sync_copy(sf_sh.at[pl.ds(s * RPT, RPT)], wb_v)
        pltpu.sync_copy(
            wb_v,
            s1_out.at[pl.ds((c * HIDDEN + f) * ACC_N + s * RPT, RPT)])


@jax.jit
def _feat_pass(src_w, dst_w, htt_flat, zeros1):
    return pl.kernel(
        _feat_kernel,
        out_type=jax.ShapeDtypeStruct((NC * HIDDEN * ACC_N,), jnp.float32),
        mesh=_mesh(),
        compiler_params=pltpu.CompilerParams(needs_layout_passes=False),
        scratch_types=[
            pltpu.VMEM((NBLK, BLK), jnp.int32),
            pltpu.VMEM((NBLK, BLK), jnp.int32),
            pltpu.VMEM((ACC_N,), jnp.float32),
            pltpu.VMEM((BLK,), jnp.float32),
            pltpu.VMEM((RPT,), jnp.float32),
            pltpu.VMEM_SHARED((ACC_N,), jnp.float32),
        ],
    )(src_w, dst_w, htt_flat, zeros1)


def _scal_kernel(src_hbm, dst_hbm, zt_hbm, zeros_hbm, s2_out,
                 src_v, dst_v, zt_v, upd_v, wb_v, s2_sh):
    c = lax.axis_index("c")
    s = lax.axis_index("s")
    wid = c * NS + s
    pltpu.sync_copy(zeros_hbm.at[pl.ds(s * RPT, RPT)], wb_v)
    pltpu.sync_copy(wb_v, s2_sh.at[pl.ds(s * RPT, RPT)])
    pltpu.sync_copy(zt_hbm, zt_v)
    pltpu.sync_copy(src_hbm.at[wid], src_v)
    pltpu.sync_copy(dst_hbm.at[wid], dst_v)
    plsc.subcore_barrier()

    def body(j, carry):
        for k in range(BLK // 16):
            idx = src_v[j, pl.ds(k * 16, 16)]
            upd_v[pl.ds(k * 16, 16)] = plsc.load_gather(zt_v, [idx])
        pltpu.sync_copy(upd_v, s2_sh.at[dst_v.at[j]], add=True)
        return carry

    lax.fori_loop(0, NBLK, body, 0)
    plsc.subcore_barrier()
    pltpu.sync_copy(s2_sh.at[pl.ds(s * RPT, RPT)], wb_v)
    pltpu.sync_copy(wb_v, s2_out.at[pl.ds(c * ACC_N + s * RPT, RPT)])


@jax.jit
def _scal_pass(src_w, dst_w, zt, zeros1):
    return pl.kernel(
        _scal_kernel,
        out_type=jax.ShapeDtypeStruct((NC * ACC_N,), jnp.float32),
        mesh=_mesh(),
        compiler_params=pltpu.CompilerParams(needs_layout_passes=False),
        scratch_types=[
            pltpu.VMEM((NBLK, BLK), jnp.int32),
            pltpu.VMEM((NBLK, BLK), jnp.int32),
            pltpu.VMEM((ACC_N,), jnp.float32),
            pltpu.VMEM((BLK,), jnp.float32),
            pltpu.VMEM((RPT,), jnp.float32),
            pltpu.VMEM_SHARED((ACC_N,), jnp.float32),
        ],
    )(src_w, dst_w, zt, zeros1)


def _mm1_body(x_ref, w_ref, o_ref):
    o_ref[...] = jnp.dot(x_ref[...], w_ref[...],
                         preferred_element_type=jnp.float32)


def _prep_body(degp_ref, ht_ref, dist_ref, htt_ref):
    deg = degp_ref[0] + degp_ref[1] + 1.0
    dist = lax.rsqrt(deg)
    dist_ref[...] = dist
    htt_ref[...] = dist * ht_ref[...]


def _mid_body(s1p_ref, dist_ref, ht_ref, b1_ref, w2_ref, zt_ref, ztt_ref):
    s1 = s1p_ref[0] + s1p_ref[1]
    dist = dist_ref[...]
    aggt = dist * s1 + (dist * dist) * ht_ref[...] + b1_ref[...]
    h2t = jnp.maximum(aggt, 0.0)
    zt_row = jnp.dot(w2_ref[...], h2t,
                     preferred_element_type=jnp.float32)
    zt_ref[...] = zt_row
    ztt_ref[...] = dist * zt_row


def _fin_body(s2p_ref, dist_ref, z_ref, b2_ref, o_ref):
    s2 = s2p_ref[0] + s2p_ref[1]
    dist = dist_ref[...]
    o_ref[...] = dist * s2 + (dist * dist) * z_ref[...] + b2_ref[...]


def _tc_call(body, out_shape, *args):
    return pl.pallas_call(body, out_shape=out_shape)(*args)


@jax.jit
def kernel(x, edge_index, W1, b1, W2, b2):
    ei = edge_index.astype(jnp.int32)
    n_pad = E_PAD - N_EDGES
    pad_i = jnp.arange(n_pad, dtype=jnp.int32)
    src_w = jnp.concatenate([ei[0], pad_i % N_NODES]).reshape(NW, NBLK, BLK)
    dst_w = jnp.concatenate(
        [ei[1], N_NODES + (pad_i % PAD_SPREAD)]).reshape(NW, NBLK, BLK)

    x_pad = jnp.pad(x, ((0, ACC_N - N_NODES), (0, 0)))
    zeros1 = jnp.zeros((ACC_N,), jnp.float32)

    h = _tc_call(_mm1_body, jax.ShapeDtypeStruct((ACC_N, HIDDEN), jnp.float32),
                 x_pad, W1)
    deg_p = _deg_pass(dst_w, zeros1)

    ht_T = h.T
    dist, htt = _tc_call(
        _prep_body,
        (jax.ShapeDtypeStruct((1, ACC_N), jnp.float32),
         jax.ShapeDtypeStruct((HIDDEN, ACC_N), jnp.float32)),
        deg_p.reshape(NC, 1, ACC_N), ht_T)

    s1_p = _feat_pass(src_w, dst_w, htt.reshape(-1), zeros1)

    z_row, zt_row = _tc_call(
        _mid_body,
        (jax.ShapeDtypeStruct((1, ACC_N), jnp.float32),
         jax.ShapeDtypeStruct((1, ACC_N), jnp.float32)),
        s1_p.reshape(NC, HIDDEN, ACC_N), dist, ht_T,
        b1.reshape(HIDDEN, 1), W2.reshape(1, HIDDEN))

    s2_p = _scal_pass(src_w, dst_w, zt_row.reshape(ACC_N), zeros1)

    outf = _tc_call(
        _fin_body, jax.ShapeDtypeStruct((1, ACC_N), jnp.float32),
        s2_p.reshape(NC, 1, ACC_N), dist, z_row, b2.reshape(1, 1))
    return outf.reshape(ACC_N, 1)[:N_NODES]


# speedup vs baseline: 38.7787x; 1.3226x over previous
import functools

import jax
import jax.numpy as jnp
from jax import lax
from jax.experimental import pallas as pl
from jax.experimental.pallas import tpu as pltpu
from jax.experimental.pallas import tpu_sc as plsc

N_NODES = 10000
N_EDGES = 320000
D_FEAT = 128
HIDDEN = 16

NC = 2
NS = 16
NW = NC * NS
BLK = 128
NBLK = 79
EPT = BLK * NBLK
E_PAD = EPT * NW
ACC_N = 10112
RPT = ACC_N // NS
PAD_SPREAD = 100


def _mesh():
    return plsc.VectorSubcoreMesh(core_axis_name="c", subcore_axis_name="s")


def _deg_kernel(dst_hbm, zeros_hbm, deg_out, dst_v, ones_v, wb_v, deg_sh):
    c = lax.axis_index("c")
    s = lax.axis_index("s")
    wid = c * NS + s
    pltpu.sync_copy(zeros_hbm.at[pl.ds(s * RPT, RPT)], wb_v)
    pltpu.sync_copy(wb_v, deg_sh.at[pl.ds(s * RPT, RPT)])
    for k in range(BLK // 16):
        ones_v[pl.ds(k * 16, 16)] = jnp.ones((16,), jnp.float32)
    pltpu.sync_copy(dst_hbm.at[wid], dst_v)
    plsc.subcore_barrier()

    def body(j, carry):
        pltpu.sync_copy(ones_v, deg_sh.at[dst_v.at[j]], add=True)
        return carry

    lax.fori_loop(0, NBLK, body, 0)
    plsc.subcore_barrier()
    pltpu.sync_copy(deg_sh.at[pl.ds(s * RPT, RPT)], wb_v)
    pltpu.sync_copy(wb_v, deg_out.at[pl.ds(c * ACC_N + s * RPT, RPT)])


@functools.partial(jax.jit, donate_argnums=())
def _deg_pass(dst_w, zeros1):
    return pl.kernel(
        _deg_kernel,
        out_type=jax.ShapeDtypeStruct((NC * ACC_N,), jnp.float32),
        mesh=_mesh(),
        scratch_types=[
            pltpu.VMEM((NBLK, BLK), jnp.int32),
            pltpu.VMEM((BLK,), jnp.float32),
            pltpu.VMEM((RPT,), jnp.float32),
            pltpu.VMEM_SHARED((ACC_N,), jnp.float32),
        ],
    )(dst_w, zeros1)


def _feat_kernel(src_hbm, dst_hbm, htt_hbm, zeros_hbm, s1_out,
                 src_v, dst_v, tab_v, upd0_v, upd1_v, wb_v, sf_sh,
                 sem0, sem1):
    c = lax.axis_index("c")
    s = lax.axis_index("s")
    wid = c * NS + s
    pltpu.sync_copy(src_hbm.at[wid], src_v)
    pltpu.sync_copy(dst_hbm.at[wid], dst_v)

    def gather(j, upd):
        for k in range(BLK // 16):
            idx = src_v[j, pl.ds(k * 16, 16)]
            upd[pl.ds(k * 16, 16)] = plsc.load_gather(tab_v, [idx])

    for f in range(HIDDEN):
        pltpu.sync_copy(zeros_hbm.at[pl.ds(s * RPT, RPT)], wb_v)
        pltpu.sync_copy(wb_v, sf_sh.at[pl.ds(s * RPT, RPT)])
        pltpu.sync_copy(htt_hbm.at[pl.ds(f * ACC_N, ACC_N)], tab_v)
        plsc.subcore_barrier()

        gather(0, upd0_v)

        def body(p, carry):
            b0 = 2 * p
            cp0 = pltpu.async_copy(upd0_v, sf_sh.at[dst_v.at[b0]], sem0,
                                   add=True)
            gather(b0 + 1, upd1_v)
            cp0.wait()
            cp1 = pltpu.async_copy(upd1_v, sf_sh.at[dst_v.at[b0 + 1]], sem1,
                                   add=True)
            gather(b0 + 2, upd0_v)
            cp1.wait()
            return carry

        lax.fori_loop(0, (NBLK - 1) // 2, body, 0)
        pltpu.sync_copy(upd0_v, sf_sh.at[dst_v.at[NBLK - 1]], add=True)
        plsc.subcore_barrier()
        pltpu.sync_copy(sf_sh.at[pl.ds(s * RPT, RPT)], wb_v)
        pltpu.sync_copy(
            wb_v,
            s1_out.at[pl.ds((c * HIDDEN + f) * ACC_N + s * RPT, RPT)])


@jax.jit
def _feat_pass(src_w, dst_w, htt_flat, zeros1):
    return pl.kernel(
        _feat_kernel,
        out_type=jax.ShapeDtypeStruct((NC * HIDDEN * ACC_N,), jnp.float32),
        mesh=_mesh(),
        compiler_params=pltpu.CompilerParams(needs_layout_passes=False),
        scratch_types=[
            pltpu.VMEM((NBLK, BLK), jnp.int32),
            pltpu.VMEM((NBLK, BLK), jnp.int32),
            pltpu.VMEM((ACC_N,), jnp.float32),
            pltpu.VMEM((BLK,), jnp.float32),
            pltpu.VMEM((BLK,), jnp.float32),
            pltpu.VMEM((RPT,), jnp.float32),
            pltpu.VMEM_SHARED((ACC_N,), jnp.float32),
            pltpu.SemaphoreType.DMA,
            pltpu.SemaphoreType.DMA,
        ],
    )(src_w, dst_w, htt_flat, zeros1)


def _scal_kernel(src_hbm, dst_hbm, zt_hbm, zeros_hbm, s2_out,
                 src_v, dst_v, zt_v, upd_v, wb_v, s2_sh):
    c = lax.axis_index("c")
    s = lax.axis_index("s")
    wid = c * NS + s
    pltpu.sync_copy(zeros_hbm.at[pl.ds(s * RPT, RPT)], wb_v)
    pltpu.sync_copy(wb_v, s2_sh.at[pl.ds(s * RPT, RPT)])
    pltpu.sync_copy(zt_hbm, zt_v)
    pltpu.sync_copy(src_hbm.at[wid], src_v)
    pltpu.sync_copy(dst_hbm.at[wid], dst_v)
    plsc.subcore_barrier()

    def body(j, carry):
        for k in range(BLK // 16):
            idx = src_v[j, pl.ds(k * 16, 16)]
            upd_v[pl.ds(k * 16, 16)] = plsc.load_gather(zt_v, [idx])
        pltpu.sync_copy(upd_v, s2_sh.at[dst_v.at[j]], add=True)
        return carry

    lax.fori_loop(0, NBLK, body, 0)
    plsc.subcore_barrier()
    pltpu.sync_copy(s2_sh.at[pl.ds(s * RPT, RPT)], wb_v)
    pltpu.sync_copy(wb_v, s2_out.at[pl.ds(c * ACC_N + s * RPT, RPT)])


@jax.jit
def _scal_pass(src_w, dst_w, zt, zeros1):
    return pl.kernel(
        _scal_kernel,
        out_type=jax.ShapeDtypeStruct((NC * ACC_N,), jnp.float32),
        mesh=_mesh(),
        compiler_params=pltpu.CompilerParams(needs_layout_passes=False),
        scratch_types=[
            pltpu.VMEM((NBLK, BLK), jnp.int32),
            pltpu.VMEM((NBLK, BLK), jnp.int32),
            pltpu.VMEM((ACC_N,), jnp.float32),
            pltpu.VMEM((BLK,), jnp.float32),
            pltpu.VMEM((RPT,), jnp.float32),
            pltpu.VMEM_SHARED((ACC_N,), jnp.float32),
        ],
    )(src_w, dst_w, zt, zeros1)


def _mm1_body(x_ref, w_ref, o_ref):
    o_ref[...] = jnp.dot(x_ref[...], w_ref[...],
                         preferred_element_type=jnp.float32)


def _prep_body(degp_ref, ht_ref, dist_ref, htt_ref):
    deg = degp_ref[0] + degp_ref[1] + 1.0
    dist = lax.rsqrt(deg)
    dist_ref[...] = dist
    htt_ref[...] = dist * ht_ref[...]


def _mid_body(s1p_ref, dist_ref, ht_ref, b1_ref, w2_ref, zt_ref, ztt_ref):
    s1 = s1p_ref[0] + s1p_ref[1]
    dist = dist_ref[...]
    aggt = dist * s1 + (dist * dist) * ht_ref[...] + b1_ref[...]
    h2t = jnp.maximum(aggt, 0.0)
    zt_row = jnp.dot(w2_ref[...], h2t,
                     preferred_element_type=jnp.float32)
    zt_ref[...] = zt_row
    ztt_ref[...] = dist * zt_row


def _fin_body(s2p_ref, dist_ref, z_ref, b2_ref, o_ref):
    s2 = s2p_ref[0] + s2p_ref[1]
    dist = dist_ref[...]
    o_ref[...] = dist * s2 + (dist * dist) * z_ref[...] + b2_ref[...]


def _tc_call(body, out_shape, *args):
    return pl.pallas_call(body, out_shape=out_shape)(*args)


@jax.jit
def kernel(x, edge_index, W1, b1, W2, b2):
    ei = edge_index.astype(jnp.int32)
    n_pad = E_PAD - N_EDGES
    pad_i = jnp.arange(n_pad, dtype=jnp.int32)
    src_w = jnp.concatenate([ei[0], pad_i % N_NODES]).reshape(NW, NBLK, BLK)
    dst_w = jnp.concatenate(
        [ei[1], N_NODES + (pad_i % PAD_SPREAD)]).reshape(NW, NBLK, BLK)

    x_pad = jnp.pad(x, ((0, ACC_N - N_NODES), (0, 0)))
    zeros1 = jnp.zeros((ACC_N,), jnp.float32)

    h = _tc_call(_mm1_body, jax.ShapeDtypeStruct((ACC_N, HIDDEN), jnp.float32),
                 x_pad, W1)
    deg_p = _deg_pass(dst_w, zeros1)

    ht_T = h.T
    dist, htt = _tc_call(
        _prep_body,
        (jax.ShapeDtypeStruct((1, ACC_N), jnp.float32),
         jax.ShapeDtypeStruct((HIDDEN, ACC_N), jnp.float32)),
        deg_p.reshape(NC, 1, ACC_N), ht_T)

    s1_p = _feat_pass(src_w, dst_w, htt.reshape(-1), zeros1)

    z_row, zt_row = _tc_call(
        _mid_body,
        (jax.ShapeDtypeStruct((1, ACC_N), jnp.float32),
         jax.ShapeDtypeStruct((1, ACC_N), jnp.float32)),
        s1_p.reshape(NC, HIDDEN, ACC_N), dist, ht_T,
        b1.reshape(HIDDEN, 1), W2.reshape(1, HIDDEN))

    s2_p = _scal_pass(src_w, dst_w, zt_row.reshape(ACC_N), zeros1)

    outf = _tc_call(
        _fin_body, jax.ShapeDtypeStruct((1, ACC_N), jnp.float32),
        s2_p.reshape(NC, 1, ACC_N), dist, z_row, b2.reshape(1, 1))
    return outf.reshape(ACC_N, 1)[:N_NODES]


# speedup vs baseline: 39.7004x; 1.0238x over previous
import functools

import jax
import jax.numpy as jnp
from jax import lax
from jax.experimental import pallas as pl
from jax.experimental.pallas import tpu as pltpu
from jax.experimental.pallas import tpu_sc as plsc

N_NODES = 10000
N_EDGES = 320000
D_FEAT = 128
HIDDEN = 16

NC = 2
NS = 16
NW = NC * NS
BLK = 128
NBLK = 79
EPT = BLK * NBLK
E_PAD = EPT * NW
ACC_N = 10112
RPT = ACC_N // NS
PAD_SPREAD = 100


def _mesh():
    return plsc.VectorSubcoreMesh(core_axis_name="c", subcore_axis_name="s")


def _deg_kernel(dst_hbm, zeros_hbm, deg_out, dst_v, ones_v, wb_v, deg_sh):
    c = lax.axis_index("c")
    s = lax.axis_index("s")
    wid = c * NS + s
    pltpu.sync_copy(zeros_hbm.at[pl.ds(s * RPT, RPT)], wb_v)
    pltpu.sync_copy(wb_v, deg_sh.at[pl.ds(s * RPT, RPT)])
    for k in range(BLK // 16):
        ones_v[pl.ds(k * 16, 16)] = jnp.ones((16,), jnp.float32)
    pltpu.sync_copy(dst_hbm.at[wid], dst_v)
    plsc.subcore_barrier()

    def body(j, carry):
        pltpu.sync_copy(ones_v, deg_sh.at[dst_v.at[j]], add=True)
        return carry

    lax.fori_loop(0, NBLK, body, 0)
    plsc.subcore_barrier()
    pltpu.sync_copy(deg_sh.at[pl.ds(s * RPT, RPT)], wb_v)
    pltpu.sync_copy(wb_v, deg_out.at[pl.ds(c * ACC_N + s * RPT, RPT)])


@functools.partial(jax.jit, donate_argnums=())
def _deg_pass(dst_w, zeros1):
    return pl.kernel(
        _deg_kernel,
        out_type=jax.ShapeDtypeStruct((NC * ACC_N,), jnp.float32),
        mesh=_mesh(),
        scratch_types=[
            pltpu.VMEM((NBLK, BLK), jnp.int32),
            pltpu.VMEM((BLK,), jnp.float32),
            pltpu.VMEM((RPT,), jnp.float32),
            pltpu.VMEM_SHARED((ACC_N,), jnp.float32),
        ],
    )(dst_w, zeros1)


def _feat_kernel(src_hbm, dst_hbm, htt_hbm, zeros_hbm, s1_out,
                 src_v, dst_v, tab_v, upd0_v, upd1_v, wb_v, sf_sh,
                 sem0, sem1):
    c = lax.axis_index("c")
    s = lax.axis_index("s")
    wid = c * NS + s
    pltpu.sync_copy(src_hbm.at[wid], src_v)
    pltpu.sync_copy(dst_hbm.at[wid], dst_v)

    def gather(j, upd):
        for k in range(BLK // 16):
            idx = src_v[j, pl.ds(k * 16, 16)]
            upd[pl.ds(k * 16, 16)] = plsc.load_gather(tab_v, [idx])

    for f in range(HIDDEN):
        pltpu.sync_copy(zeros_hbm.at[pl.ds(s * RPT, RPT)], wb_v)
        pltpu.sync_copy(wb_v, sf_sh.at[pl.ds(s * RPT, RPT)])
        pltpu.sync_copy(htt_hbm.at[pl.ds(f * ACC_N, ACC_N)], tab_v)
        plsc.subcore_barrier()

        gather(0, upd0_v)

        def body(p, carry):
            b0 = 2 * p
            cp0 = pltpu.async_copy(upd0_v, sf_sh.at[dst_v.at[b0]], sem0,
                                   add=True)
            gather(b0 + 1, upd1_v)
            cp0.wait()
            cp1 = pltpu.async_copy(upd1_v, sf_sh.at[dst_v.at[b0 + 1]], sem1,
                                   add=True)
            gather(b0 + 2, upd0_v)
            cp1.wait()
            return carry

        lax.fori_loop(0, (NBLK - 1) // 2, body, 0)
        pltpu.sync_copy(upd0_v, sf_sh.at[dst_v.at[NBLK - 1]], add=True)
        plsc.subcore_barrier()
        pltpu.sync_copy(sf_sh.at[pl.ds(s * RPT, RPT)], wb_v)
        pltpu.sync_copy(
            wb_v,
            s1_out.at[pl.ds((c * HIDDEN + f) * ACC_N + s * RPT, RPT)])


@jax.jit
def _feat_pass(src_w, dst_w, htt_flat, zeros1):
    return pl.kernel(
        _feat_kernel,
        out_type=jax.ShapeDtypeStruct((NC * HIDDEN * ACC_N,), jnp.float32),
        mesh=_mesh(),
        compiler_params=pltpu.CompilerParams(needs_layout_passes=False),
        scratch_types=[
            pltpu.VMEM((NBLK, BLK), jnp.int32),
            pltpu.VMEM((NBLK, BLK), jnp.int32),
            pltpu.VMEM((ACC_N,), jnp.float32),
            pltpu.VMEM((BLK,), jnp.float32),
            pltpu.VMEM((BLK,), jnp.float32),
            pltpu.VMEM((RPT,), jnp.float32),
            pltpu.VMEM_SHARED((ACC_N,), jnp.float32),
            pltpu.SemaphoreType.DMA,
            pltpu.SemaphoreType.DMA,
        ],
    )(src_w, dst_w, htt_flat, zeros1)


def _scal_kernel(src_hbm, dst_hbm, zt_hbm, zeros_hbm, s2_out,
                 src_v, dst_v, zt_v, upd0_v, upd1_v, wb_v, s2_sh,
                 sem0, sem1):
    c = lax.axis_index("c")
    s = lax.axis_index("s")
    wid = c * NS + s
    pltpu.sync_copy(zeros_hbm.at[pl.ds(s * RPT, RPT)], wb_v)
    pltpu.sync_copy(wb_v, s2_sh.at[pl.ds(s * RPT, RPT)])
    pltpu.sync_copy(zt_hbm, zt_v)
    pltpu.sync_copy(src_hbm.at[wid], src_v)
    pltpu.sync_copy(dst_hbm.at[wid], dst_v)
    plsc.subcore_barrier()

    def gather(j, upd):
        for k in range(BLK // 16):
            idx = src_v[j, pl.ds(k * 16, 16)]
            upd[pl.ds(k * 16, 16)] = plsc.load_gather(zt_v, [idx])

    gather(0, upd0_v)

    def body(p, carry):
        b0 = 2 * p
        cp0 = pltpu.async_copy(upd0_v, s2_sh.at[dst_v.at[b0]], sem0, add=True)
        gather(b0 + 1, upd1_v)
        cp0.wait()
        cp1 = pltpu.async_copy(upd1_v, s2_sh.at[dst_v.at[b0 + 1]], sem1,
                               add=True)
        gather(b0 + 2, upd0_v)
        cp1.wait()
        return carry

    lax.fori_loop(0, (NBLK - 1) // 2, body, 0)
    pltpu.sync_copy(upd0_v, s2_sh.at[dst_v.at[NBLK - 1]], add=True)
    plsc.subcore_barrier()
    pltpu.sync_copy(s2_sh.at[pl.ds(s * RPT, RPT)], wb_v)
    pltpu.sync_copy(wb_v, s2_out.at[pl.ds(c * ACC_N + s * RPT, RPT)])


@jax.jit
def _scal_pass(src_w, dst_w, zt, zeros1):
    return pl.kernel(
        _scal_kernel,
        out_type=jax.ShapeDtypeStruct((NC * ACC_N,), jnp.float32),
        mesh=_mesh(),
        compiler_params=pltpu.CompilerParams(needs_layout_passes=False),
        scratch_types=[
            pltpu.VMEM((NBLK, BLK), jnp.int32),
            pltpu.VMEM((NBLK, BLK), jnp.int32),
            pltpu.VMEM((ACC_N,), jnp.float32),
            pltpu.VMEM((BLK,), jnp.float32),
            pltpu.VMEM((BLK,), jnp.float32),
            pltpu.VMEM((RPT,), jnp.float32),
            pltpu.VMEM_SHARED((ACC_N,), jnp.float32),
            pltpu.SemaphoreType.DMA,
            pltpu.SemaphoreType.DMA,
        ],
    )(src_w, dst_w, zt, zeros1)


def _mm1_body(x_ref, w_ref, o_ref):
    o_ref[...] = jnp.dot(x_ref[...], w_ref[...],
                         preferred_element_type=jnp.float32)


def _prep_body(degp_ref, ht_ref, dist_ref, htt_ref):
    deg = degp_ref[0] + degp_ref[1] + 1.0
    dist = lax.rsqrt(deg)
    dist_ref[...] = dist
    htt_ref[...] = dist * ht_ref[...]


def _mid_body(s1p_ref, dist_ref, ht_ref, b1_ref, w2_ref, zt_ref, ztt_ref):
    s1 = s1p_ref[0] + s1p_ref[1]
    dist = dist_ref[...]
    aggt = dist * s1 + (dist * dist) * ht_ref[...] + b1_ref[...]
    h2t = jnp.maximum(aggt, 0.0)
    zt_row = jnp.dot(w2_ref[...], h2t,
                     preferred_element_type=jnp.float32)
    zt_ref[...] = zt_row
    ztt_ref[...] = dist * zt_row


def _fin_body(s2p_ref, dist_ref, z_ref, b2_ref, o_ref):
    s2 = s2p_ref[0] + s2p_ref[1]
    dist = dist_ref[...]
    o_ref[...] = dist * s2 + (dist * dist) * z_ref[...] + b2_ref[...]


def _tc_call(body, out_shape, *args):
    return pl.pallas_call(body, out_shape=out_shape)(*args)


@jax.jit
def kernel(x, edge_index, W1, b1, W2, b2):
    ei = edge_index.astype(jnp.int32)
    n_pad = E_PAD - N_EDGES
    pad_i = jnp.arange(n_pad, dtype=jnp.int32)
    src_w = jnp.concatenate([ei[0], pad_i % N_NODES]).reshape(NW, NBLK, BLK)
    dst_w = jnp.concatenate(
        [ei[1], N_NODES + (pad_i % PAD_SPREAD)]).reshape(NW, NBLK, BLK)

    x_pad = jnp.pad(x, ((0, ACC_N - N_NODES), (0, 0)))
    zeros1 = jnp.zeros((ACC_N,), jnp.float32)

    h = _tc_call(_mm1_body, jax.ShapeDtypeStruct((ACC_N, HIDDEN), jnp.float32),
                 x_pad, W1)
    deg_p = _deg_pass(dst_w, zeros1)

    ht_T = h.T
    dist, htt = _tc_call(
        _prep_body,
        (jax.ShapeDtypeStruct((1, ACC_N), jnp.float32),
         jax.ShapeDtypeStruct((HIDDEN, ACC_N), jnp.float32)),
        deg_p.reshape(NC, 1, ACC_N), ht_T)

    s1_p = _feat_pass(src_w, dst_w, htt.reshape(-1), zeros1)

    z_row, zt_row = _tc_call(
        _mid_body,
        (jax.ShapeDtypeStruct((1, ACC_N), jnp.float32),
         jax.ShapeDtypeStruct((1, ACC_N), jnp.float32)),
        s1_p.reshape(NC, HIDDEN, ACC_N), dist, ht_T,
        b1.reshape(HIDDEN, 1), W2.reshape(1, HIDDEN))

    s2_p = _scal_pass(src_w, dst_w, zt_row.reshape(ACC_N), zeros1)

    outf = _tc_call(
        _fin_body, jax.ShapeDtypeStruct((1, ACC_N), jnp.float32),
        s2_p.reshape(NC, 1, ACC_N), dist, z_row, b2.reshape(1, 1))
    return outf.reshape(ACC_N, 1)[:N_NODES]
